# baseline (speedup 1.0000x reference)
import functools

import jax
import jax.numpy as jnp
from jax import lax
from jax.experimental import pallas as pl
from jax.experimental.pallas import tpu as pltpu
from jax.experimental.pallas import tpu_sc as plsc

_INFO = plsc.get_sparse_core_info()
_NC = _INFO.num_cores
_NS = _INFO.num_subcores
_NW = _NC * _NS


@functools.lru_cache(maxsize=None)
def _make_kernel(jh: int, ih: int):
    assert ih % _NW == 0, ih
    ih_per_worker = ih // _NW
    rows = 8 * jh
    assert rows % 2 == 0
    half = rows // 2
    mesh = plsc.VectorSubcoreMesh(core_axis_name="c", subcore_axis_name="s")

    @functools.partial(
        pl.kernel,
        mesh=mesh,
        compiler_params=pltpu.CompilerParams(
            needs_layout_passes=False,
            use_tc_tiling_on_sc=False,
        ),
        out_type=jax.ShapeDtypeStruct((8 * jh, ih, 2, 128), jnp.float32),
        scratch_types=[
            pltpu.VMEM((16,), jnp.float32),
            pltpu.VMEM((16,), jnp.float32),
            pltpu.VMEM((16,), jnp.float32),
            pltpu.VMEM((jh, 8, 128), jnp.int32),
            pltpu.VMEM((jh, 8, 128), jnp.int32),
            pltpu.VMEM((half, 2, 128), jnp.float32),
            pltpu.VMEM((half, 2, 128), jnp.float32),
            pltpu.SemaphoreType.DMA,
            pltpu.SemaphoreType.DMA,
            pltpu.SemaphoreType.DMA,
            pltpu.SemaphoreType.DMA,
        ],
    )
    def sc_kernel(x_hbm, params_hbm, out_hbm, params_v, tab0_v, tab1_v,
                  x_v0, x_v1, out_v0, out_v1, isem0, isem1, osem0, osem1):
        wid = lax.axis_index("s") * _NC + lax.axis_index("c")
        lane = lax.iota(jnp.int32, 16)

        pltpu.sync_copy(params_hbm, params_v)
        v4 = lane & 3
        e0 = plsc.load_gather(params_v, [v4 * 2])
        e1 = plsc.load_gather(params_v, [v4 * 2 + 1])

        def splat(i):
            return plsc.load_gather(params_v, [jnp.full((16,), i, jnp.int32)])

        tab0_v[...] = e0 * splat(8) + e1 * splat(9) + splat(12)
        tab1_v[...] = e0 * splat(10) + e1 * splat(11) + splat(13)

        x_bufs = (x_v0, x_v1)
        out_bufs = (out_v0, out_v1)
        isems = (isem0, isem1)
        osems = (osem0, osem1)

        def start_in(u):
            return pltpu.async_copy(
                x_hbm.at[pl.ds(0, jh), wid + u * _NW], x_bufs[u & 1], isems[u & 1]
            )

        in_cp = start_in(0)
        out_cps = {}
        for u in range(ih_per_worker):
            next_cp = start_in(u + 1) if u + 1 < ih_per_worker else None
            in_cp.wait()
            x_v = x_bufs[u & 1]
            for h in range(2):
                if u > 0:
                    out_cps.pop((u - 1, h)).wait()
                out_v = out_bufs[h]
                r0 = half * h

                @plsc.parallel_loop(0, half, unroll=2)
                def body(rl):
                    r = rl + r0
                    jhl = r >> 3
                    jlo = r & 7
                    for l in range(8):
                        idx = x_v[jhl, jlo, pl.ds(l * 16, 16)]
                        v0 = plsc.load_gather(tab0_v, [idx])
                        v1 = plsc.load_gather(tab1_v, [idx])
                        out_v[rl, 0, pl.ds(l * 16, 16)] = v0
                        out_v[rl, 1, pl.ds(l * 16, 16)] = v1

                out_cps[(u, h)] = pltpu.async_copy(
                    out_v,
                    out_hbm.at[pl.ds(r0, half), wid + u * _NW],
                    osems[h],
                )
            in_cp = next_cp
        for k in sorted(out_cps):
            out_cps.pop(k).wait()

    return sc_kernel


def kernel(x, emb, W, b):
    nrows, ncols = x.shape
    jh, ih = ncols // 8, nrows // 128
    params = jnp.concatenate([
        emb.reshape(-1).astype(jnp.float32),
        W.reshape(-1).astype(jnp.float32),
        b.astype(jnp.float32),
        jnp.zeros((2,), jnp.float32),
    ])
    x4 = x.astype(jnp.int32).T.reshape(jh, 8, ih, 128).transpose(0, 2, 1, 3)
    out4 = _make_kernel(jh, ih)(x4, params)
    return out4.transpose(1, 3, 0, 2).reshape(nrows, ncols, 2)


# speedup vs baseline: 1.0719x; 1.0719x over previous
import functools

import jax
import jax.numpy as jnp
from jax import lax
from jax.experimental import pallas as pl
from jax.experimental.pallas import tpu as pltpu
from jax.experimental.pallas import tpu_sc as plsc

_INFO = plsc.get_sparse_core_info()
_NC = _INFO.num_cores
_NS = _INFO.num_subcores
_NW = _NC * _NS


@functools.lru_cache(maxsize=None)
def _make_kernel(jh: int, ih: int):
    assert ih % _NW == 0, ih
    ih_per_worker = ih // _NW
    rows = 8 * jh
    assert rows % 2 == 0
    half = rows // 2
    mesh = plsc.VectorSubcoreMesh(core_axis_name="c", subcore_axis_name="s")

    @functools.partial(
        pl.kernel,
        mesh=mesh,
        compiler_params=pltpu.CompilerParams(
            needs_layout_passes=False,
            use_tc_tiling_on_sc=False,
        ),
        out_type=jax.ShapeDtypeStruct((8 * jh, ih, 2, 128), jnp.float32),
        scratch_types=[
            pltpu.VMEM((16,), jnp.float32),
            pltpu.VMEM((16,), jnp.float32),
            pltpu.VMEM((16,), jnp.float32),
            pltpu.VMEM((jh, 8, 128), jnp.int32),
            pltpu.VMEM((jh, 8, 128), jnp.int32),
            pltpu.VMEM((half, 2, 128), jnp.float32),
            pltpu.VMEM((half, 2, 128), jnp.float32),
            pltpu.SemaphoreType.DMA,
            pltpu.SemaphoreType.DMA,
            pltpu.SemaphoreType.DMA,
            pltpu.SemaphoreType.DMA,
        ],
    )
    def sc_kernel(x_hbm, params_hbm, out_hbm, params_v, tab0_v, tab1_v,
                  x_v0, x_v1, out_v0, out_v1, isem0, isem1, osem0, osem1):
        wid = lax.axis_index("s") * _NC + lax.axis_index("c")
        lane = lax.iota(jnp.int32, 16)

        pltpu.sync_copy(params_hbm, params_v)
        v4 = lane & 3
        e0 = plsc.load_gather(params_v, [v4 * 2])
        e1 = plsc.load_gather(params_v, [v4 * 2 + 1])

        def splat(i):
            return plsc.load_gather(params_v, [jnp.full((16,), i, jnp.int32)])

        tab0_v[...] = e0 * splat(8) + e1 * splat(9) + splat(12)
        tab1_v[...] = e0 * splat(10) + e1 * splat(11) + splat(13)

        x_bufs = (x_v0, x_v1)
        out_bufs = (out_v0, out_v1)
        isems = (isem0, isem1)
        osems = (osem0, osem1)

        def in_copy(u, b):
            return pltpu.make_async_copy(
                x_hbm.at[pl.ds(0, jh), wid + u * _NW], x_bufs[b], isems[b]
            )

        def out_copy(u, h):
            return pltpu.make_async_copy(
                out_bufs[h],
                out_hbm.at[pl.ds(half * h, half), wid + u * _NW],
                osems[h],
            )

        in_copy(0, 0).start()
        in_copy(1, 1).start()

        def outer(up, carry):
            for b in range(2):
                u = 2 * up + b
                in_copy(u, b).wait()
                x_v = x_bufs[b]
                for h in range(2):
                    if b == 1:
                        out_copy(u - 1, h).wait()
                    else:
                        @pl.when(up > 0)
                        def _():
                            out_copy(u - 1, h).wait()
                    out_v = out_bufs[h]
                    r0 = half * h

                    @plsc.parallel_loop(0, half, unroll=2)
                    def body(rl):
                        r = rl + r0
                        jhl = r >> 3
                        jlo = r & 7
                        for l in range(8):
                            idx = x_v[jhl, jlo, pl.ds(l * 16, 16)]
                            v0 = plsc.load_gather(tab0_v, [idx])
                            v1 = plsc.load_gather(tab1_v, [idx])
                            out_v[rl, 0, pl.ds(l * 16, 16)] = v0
                            out_v[rl, 1, pl.ds(l * 16, 16)] = v1

                    out_copy(u, h).start()

                @pl.when(up + 1 < ih_per_worker // 2)
                def _():
                    in_copy(u + 2, b).start()
            return carry

        lax.fori_loop(0, ih_per_worker // 2, outer, 0)
        for h in range(2):
            out_copy(ih_per_worker - 1, h).wait()

    return sc_kernel


def kernel(x, emb, W, b):
    nrows, ncols = x.shape
    jh, ih = ncols // 8, nrows // 128
    params = jnp.concatenate([
        emb.reshape(-1).astype(jnp.float32),
        W.reshape(-1).astype(jnp.float32),
        b.astype(jnp.float32),
        jnp.zeros((2,), jnp.float32),
    ])
    x4 = x.astype(jnp.int32).T.reshape(jh, 8, ih, 128).transpose(0, 2, 1, 3)
    out4 = _make_kernel(jh, ih)(x4, params)
    return out4.transpose(1, 3, 0, 2).reshape(nrows, ncols, 2)
